# baseline (speedup 1.0000x reference)
import jax
import jax.numpy as jnp
import numpy as np
from jax import lax
from jax.experimental import pallas as pl
from jax.experimental.pallas import tpu as pltpu
from jax.experimental.pallas import tpu_sc as plsc

B = 16384
NCAT = 26
DNUM = 13
DT = 32
NTOK = 1 + DNUM + NCAT
NC = 2
NS = 16
NW = NC * NS
BPW = B // NW
NB = 16
NCHUNK = BPW // NB
F = NB * NCAT
G = 104
NG = F // G
PERIOD = 208


def _body(xnum_hbm, xcat_hbm, w_hbm, b_hbm, table_hbm, offs_hbm, out_hbm,
          xcat_v, xnum_v, w_v, b_v, offs_v,
          idx_v, temp_v, obuf_v, sem, osem):
    cid = lax.axis_index("c")
    sid = lax.axis_index("s")
    wid = sid * NC + cid
    bb0 = wid * BPW
    fb0 = bb0 * NCAT

    pltpu.sync_copy(xcat_hbm.at[pl.ds(fb0, BPW * NCAT)], xcat_v)
    pltpu.sync_copy(xnum_hbm.at[:, pl.ds(bb0, BPW)], xnum_v)
    pltpu.sync_copy(w_hbm, w_v)
    pltpu.sync_copy(b_hbm, b_v)
    pltpu.sync_copy(offs_hbm, offs_v)

    def chunk(t, carry):
        f0 = t * F
        buf = lax.rem(t, 2)

        @pl.when(t >= 2)
        def _():
            pltpu.make_async_copy(
                obuf_v.at[buf], out_hbm.at[pl.ds(bb0 + (t - 2) * NB, NB)],
                osem).wait()

        for o in range(0, F, 16):
            idx_v[pl.ds(o, 16)] = (xcat_v[pl.ds(f0 + o, 16)]
                                   + offs_v[pl.ds(o % PERIOD, 16)])

        handles = [
            pltpu.async_copy(table_hbm.at[idx_v.at[pl.ds(g * G, G)]],
                             temp_v.at[pl.ds(g * G, G)], sem)
            for g in range(NG)
        ]

        xvs = [xnum_v[j, pl.ds(t * NB, 16)] for j in range(DNUM)]
        for bb in range(NB):
            for h2 in range(2):
                sl = pl.ds(h2 * 16, 16)
                obuf_v[buf, bb, 0, sl] = w_v[0, sl]
            for j in range(1, DNUM + 1):
                xs = xvs[j - 1][bb]
                for h2 in range(2):
                    sl = pl.ds(h2 * 16, 16)
                    obuf_v[buf, bb, j, sl] = w_v[j, sl] * xs + b_v[j - 1, sl]

        for h in handles:
            h.wait()

        def biasb(bb, c2):
            r = bb * NCAT
            for c in range(NCAT):
                for h2 in range(2):
                    sl = pl.ds(h2 * 16, 16)
                    obuf_v[buf, bb, 1 + DNUM + c, sl] = (
                        temp_v[r + c, sl] + b_v[13 + c, sl])
            return c2

        lax.fori_loop(0, NB, biasb, 0)

        pltpu.async_copy(obuf_v.at[buf],
                         out_hbm.at[pl.ds(bb0 + t * NB, NB)], osem)
        return carry

    lax.fori_loop(0, NCHUNK, chunk, 0)
    for u in range(2):
        t = NCHUNK - 2 + u
        pltpu.make_async_copy(
            obuf_v.at[t % 2], out_hbm.at[pl.ds(bb0 + t * NB, NB)],
            osem).wait()


def kernel(x_num, x_cat, weight, bias, emb_table, category_offsets):
    xcat_flat = x_cat.reshape(-1)
    offs_pat = jnp.tile(category_offsets, PERIOD // NCAT)

    kfn = pl.kernel(
        _body,
        out_type=jax.ShapeDtypeStruct((B, NTOK, DT), jnp.float32),
        mesh=plsc.VectorSubcoreMesh(core_axis_name="c", subcore_axis_name="s"),
        compiler_params=pltpu.CompilerParams(use_tc_tiling_on_sc=False),
        scratch_types=[
            pltpu.VMEM((BPW * NCAT,), jnp.int32),
            pltpu.VMEM((DNUM, BPW), jnp.float32),
            pltpu.VMEM((DNUM + 1, DT), jnp.float32),
            pltpu.VMEM((DNUM + NCAT, DT), jnp.float32),
            pltpu.VMEM((PERIOD,), jnp.int32),
            pltpu.VMEM((F,), jnp.int32),
            pltpu.VMEM((F, DT), jnp.float32),
            pltpu.VMEM((2, NB, NTOK, DT), jnp.float32),
            pltpu.SemaphoreType.DMA,
            pltpu.SemaphoreType.DMA,
        ],
    )
    return kfn(x_num.T, xcat_flat, weight, bias, emb_table, offs_pat)
